# baseline (speedup 1.0000x reference)
import functools

import jax
import jax.numpy as jnp
from jax import lax
from jax.experimental import pallas as pl
from jax.experimental.pallas import tpu as pltpu
from jax.experimental.pallas import tpu_sc as plsc

_EMBED = 64


@functools.lru_cache(maxsize=None)
def _build(B, S, D):
    info = plsc.get_sparse_core_info()
    NC, NS, L = info.num_cores, info.num_subcores, info.num_lanes
    NW = NC * NS
    N = B * S
    RPW = N // NW
    CH = 128
    NCH = RPW // CH
    assert RPW * NW == N and CH * NCH == RPW and S % RPW == 0 and D % L == 0

    mesh = plsc.VectorSubcoreMesh(core_axis_name="c", subcore_axis_name="s")

    @functools.partial(
        pl.kernel,
        mesh=mesh,
        out_type=jax.ShapeDtypeStruct((N, D), jnp.float32),
        compiler_params=pltpu.CompilerParams(use_tc_tiling_on_sc=False),
        scratch_types=[
            pltpu.VMEM((NCH, CH), jnp.int32),
            pltpu.VMEM((RPW, D), jnp.float32),
            pltpu.VMEM((RPW, D), jnp.float32),
            [pltpu.SemaphoreType.DMA for _ in range(NCH)],
        ],
    )
    def emb_kernel(idx_hbm, tok_hbm, pos_hbm, out_hbm, idx_v, rows_v, pos_v, sems):
        wid = lax.axis_index("s") * NC + lax.axis_index("c")
        base = wid * RPW
        pbase = lax.rem(wid, S // RPW) * RPW
        for j in range(NCH):
            pltpu.sync_copy(idx_hbm.at[pl.ds(base + j * CH, CH)], idx_v.at[j])
        cps = [
            pltpu.async_copy(
                tok_hbm.at[idx_v.at[j]], rows_v.at[pl.ds(j * CH, CH)], sems[j]
            )
            for j in range(NCH)
        ]
        pltpu.sync_copy(pos_hbm.at[pl.ds(pbase, RPW)], pos_v)
        for j in range(NCH):
            cps[j].wait()

            def body(r, _):
                for c in range(D // L):
                    sl = pl.ds(c * L, L)
                    rows_v[r, sl] = rows_v[r, sl] + pos_v[r, sl]
                return 0

            lax.fori_loop(j * CH, (j + 1) * CH, body, 0)
        pltpu.sync_copy(rows_v, out_hbm.at[pl.ds(base, RPW)])

    return emb_kernel


def kernel(inputs, token_table, pos_table):
    B, S = inputs.shape
    idx = inputs.reshape(-1)
    out = _build(B, S, _EMBED)(idx, token_table, pos_table)
    return out.reshape(B, S, _EMBED)


# speedup vs baseline: 1.6960x; 1.6960x over previous
import functools

import jax
import jax.numpy as jnp
from jax import lax
from jax.experimental import pallas as pl
from jax.experimental.pallas import tpu as pltpu
from jax.experimental.pallas import tpu_sc as plsc

_EMBED = 64


@functools.lru_cache(maxsize=None)
def _build(B, S, D):
    info = plsc.get_sparse_core_info()
    NC, NS, L = info.num_cores, info.num_subcores, info.num_lanes
    NW = NC * NS
    N = B * S
    RPW = N // NW
    G = RPW // L
    assert RPW * NW == N and G * L == RPW and S % RPW == 0 and D % L == 0

    mesh = plsc.VectorSubcoreMesh(core_axis_name="c", subcore_axis_name="s")

    @functools.partial(
        pl.kernel,
        mesh=mesh,
        out_type=jax.ShapeDtypeStruct((N, D), jnp.float32),
        scratch_types=[
            pltpu.VMEM((RPW,), jnp.int32),
            pltpu.VMEM((RPW, D), jnp.float32),
            pltpu.VMEM((RPW, D), jnp.float32),
            pltpu.SemaphoreType.DMA,
            pltpu.SemaphoreType.DMA,
        ],
    )
    def emb_kernel(idx_hbm, tok_hbm, pos_hbm, out_hbm, idx_v, rows_v, pos_v,
                   gsem, psem):
        wid = lax.axis_index("s") * NC + lax.axis_index("c")
        base = wid * RPW
        pbase = lax.rem(wid, S // RPW) * RPW
        pltpu.sync_copy(idx_hbm.at[pl.ds(base, RPW)], idx_v)
        pcp = pltpu.async_copy(pos_hbm.at[pl.ds(pbase, RPW)], pos_v, psem)

        def enqueue(g, _):
            vals = idx_v[pl.ds(g * L, L)]
            for l in range(L):
                t = vals[l]
                pltpu.async_copy(
                    tok_hbm.at[pl.ds(t, 1)],
                    rows_v.at[pl.ds(g * L + l, 1)],
                    gsem,
                )
            return 0

        lax.fori_loop(0, G, enqueue, 0)
        pltpu.make_async_copy(tok_hbm.at[pl.ds(0, RPW)], rows_v, gsem).wait()
        pcp.wait()

        def add_pos(r, _):
            for c in range(D // L):
                sl = pl.ds(c * L, L)
                rows_v[r, sl] = rows_v[r, sl] + pos_v[r, sl]
            return 0

        lax.fori_loop(0, RPW, add_pos, 0)
        pltpu.sync_copy(rows_v, out_hbm.at[pl.ds(base, RPW)])

    return emb_kernel


def kernel(inputs, token_table, pos_table):
    B, S = inputs.shape
    idx = inputs.reshape(-1)
    out = _build(B, S, _EMBED)(idx, token_table, pos_table)
    return out.reshape(B, S, _EMBED)
